# baseline (speedup 1.0000x reference)
import functools

import jax
import jax.numpy as jnp
from jax import lax
from jax.experimental import pallas as pl
from jax.experimental.pallas import tpu as pltpu
from jax.experimental.pallas import tpu_sc as plsc

_ROUTER_DIM = 80
_GATE_DIM = 16
_QUERY_DIM = _ROUTER_DIM + _GATE_DIM
_TOP_K = 4
_N_EXPERTS = 48
_TB = 4096

_DN = (((0,), (1,)), ((), ()))

_NC, _NS, _L = 2, 16, 16
_NW = _NC * _NS


def _logits_body(q_ref, wr_ref, wg_ref, b_ref, r_ref, gb_ref):
    q = q_ref[...]
    router = lax.dot_general(wr_ref[...], q[:, :_ROUTER_DIM], _DN,
                             preferred_element_type=jnp.float32)
    r_ref[...] = router + b_ref[:, 0:1]
    gb = lax.dot_general(wg_ref[...], q[:, _ROUTER_DIM:], _DN,
                         preferred_element_type=jnp.float32)
    gb_ref[...] = gb + b_ref[:, 1:2]


def _tc_logits(query, wr, wg, biases):
    n_tokens = query.shape[0]
    grid = (n_tokens // _TB,)
    out_shapes = (
        jax.ShapeDtypeStruct((_N_EXPERTS, n_tokens), jnp.float32),
        jax.ShapeDtypeStruct((_N_EXPERTS, n_tokens), jnp.float32),
    )
    return pl.pallas_call(
        _logits_body,
        grid=grid,
        in_specs=[
            pl.BlockSpec((_TB, _QUERY_DIM), lambda i: (i, 0)),
            pl.BlockSpec((_ROUTER_DIM, _N_EXPERTS), lambda i: (0, 0)),
            pl.BlockSpec((_GATE_DIM, _N_EXPERTS), lambda i: (0, 0)),
            pl.BlockSpec((_N_EXPERTS, 2), lambda i: (0, 0)),
        ],
        out_specs=(
            pl.BlockSpec((_N_EXPERTS, _TB), lambda i: (0, i)),
            pl.BlockSpec((_N_EXPERTS, _TB), lambda i: (0, i)),
        ),
        out_shape=out_shapes,
    )(query, wr, wg, biases)


def _make_sc_topk(n_tokens):
    tpw = n_tokens // _NW
    mesh = plsc.VectorSubcoreMesh(core_axis_name="c", subcore_axis_name="s")

    @functools.partial(
        pl.kernel,
        mesh=mesh,
        compiler_params=pltpu.CompilerParams(needs_layout_passes=False),
        out_type=(
            jax.ShapeDtypeStruct((_TOP_K, n_tokens), jnp.float32),
            jax.ShapeDtypeStruct((_TOP_K, n_tokens), jnp.float32),
            jax.ShapeDtypeStruct((_TOP_K, n_tokens), jnp.int32),
        ),
        scratch_types=[
            pltpu.VMEM((_N_EXPERTS, tpw), jnp.float32),
            pltpu.VMEM((_N_EXPERTS * tpw,), jnp.float32),
            pltpu.VMEM((_TOP_K, tpw), jnp.float32),
            pltpu.VMEM((_TOP_K, tpw), jnp.float32),
            pltpu.VMEM((_TOP_K, tpw), jnp.int32),
            pltpu.SemaphoreType.DMA,
        ],
    )
    def sc_topk(r_hbm, gb_hbm, rs_hbm, gs_hbm, idx_hbm,
                r_v, gb_v, rs_v, gs_v, idx_v, sem):
        wid = lax.axis_index("s") * _NC + lax.axis_index("c")
        base = wid * tpw
        copies = [
            pltpu.async_copy(gb_hbm.at[e, pl.ds(base, tpw)],
                             gb_v.at[pl.ds(e * tpw, tpw)], sem)
            for e in range(_N_EXPERTS)
        ]
        pltpu.sync_copy(r_hbm.at[:, pl.ds(base, tpw)], r_v)
        for c in copies:
            c.wait()

        neg_inf = jnp.float32(-jnp.inf)
        iota = lax.iota(jnp.int32, _L)

        def t_body(t, carry):
            off = t * _L
            m1 = jnp.full((_L,), neg_inf, jnp.float32)
            m2, m3, m4 = m1, m1, m1
            i1 = jnp.zeros((_L,), jnp.int32)
            i2, i3, i4 = i1, i1, i1
            for e in range(_N_EXPERTS):
                v = r_v[e, pl.ds(off, _L)]
                ev = jnp.full((_L,), e, jnp.int32)
                c1 = v > m1
                c2 = v > m2
                c3 = v > m3
                c4 = v > m4
                m4 = jnp.where(c3, m3, jnp.where(c4, v, m4))
                i4 = jnp.where(c3, i3, jnp.where(c4, ev, i4))
                m3 = jnp.where(c2, m2, jnp.where(c3, v, m3))
                i3 = jnp.where(c2, i2, jnp.where(c3, ev, i3))
                m2 = jnp.where(c1, m1, jnp.where(c2, v, m2))
                i2 = jnp.where(c1, i1, jnp.where(c2, ev, i2))
                m1 = jnp.where(c1, v, m1)
                i1 = jnp.where(c1, ev, i1)
            tix = off + iota
            for k, (m, i) in enumerate(
                    ((m1, i1), (m2, i2), (m3, i3), (m4, i4))):
                g = plsc.load_gather(gb_v, [i * tpw + tix])
                rs_v[k, pl.ds(off, _L)] = 1.0 / (1.0 + jnp.exp(-m))
                gs_v[k, pl.ds(off, _L)] = 1.0 / (1.0 + jnp.exp(-(m + g)))
                idx_v[k, pl.ds(off, _L)] = i
            return carry

        lax.fori_loop(0, tpw // _L, t_body, 0)

        pltpu.sync_copy(rs_v, rs_hbm.at[:, pl.ds(base, tpw)])
        pltpu.sync_copy(gs_v, gs_hbm.at[:, pl.ds(base, tpw)])
        pltpu.sync_copy(idx_v, idx_hbm.at[:, pl.ds(base, tpw)])

    return sc_topk


@jax.jit
def kernel(query, key_pool):
    kp = key_pool[0]
    wr = kp[:_ROUTER_DIM, :]
    wg = kp[_ROUTER_DIM:_QUERY_DIM, :]
    biases = jnp.stack([kp[-4, :], kp[-3, :]], axis=1)
    n_tokens = query.shape[0]
    router, gb = _tc_logits(query, wr, wg, biases)
    rs, gs, idx = _make_sc_topk(n_tokens)(router, gb)
    return rs.T, gs.T, idx.T
